# baseline (speedup 1.0000x reference)
import functools

import jax
import jax.numpy as jnp
from jax import lax
from jax.experimental import pallas as pl
from jax.experimental.pallas import tpu as pltpu

_EPS = 1e-5
_V = 25
_TB = 8
_Q = _TB * _V
_PREC = lax.Precision.DEFAULT
_PREC_G = lax.Precision.HIGHEST


def _bnfold(g, b, m, v):
    s = g / jnp.sqrt(v + _EPS)
    return s, b - s * m


def _g_kernel(x_ref, w1_ref, b1_ref, w2_ref, b2_ref, g_ref):
    x = x_ref[...]
    p = jnp.dot(w1_ref[...], x, preferred_element_type=jnp.float32,
                precision=_PREC_G) + b1_ref[...]
    q = jnp.dot(w2_ref[...], x, preferred_element_type=jnp.float32,
                precision=_PREC_G) + b2_ref[...]
    s = lax.dot_general(p, q, (((0,), (0,)), ((), ())),
                        preferred_element_type=jnp.float32,
                        precision=_PREC_G)
    for tt in range(_TB):
        sb = s[tt * _V:(tt + 1) * _V, tt * _V:(tt + 1) * _V]
        sb = sb - jnp.max(sb, axis=-1, keepdims=True)
        e = jnp.exp(sb)
        g_ref[tt] = (e / jnp.sum(e, axis=-1, keepdims=True)).astype(g_ref.dtype)


def _compute_g(xq, wg1, bg1, wg2, bg2, t):
    n, tc, cin, q = xq.shape
    dg = wg1.shape[0]
    return pl.pallas_call(
        _g_kernel,
        out_shape=jax.ShapeDtypeStruct((n, t, _V, _V), xq.dtype),
        grid=(n, tc),
        in_specs=[
            pl.BlockSpec((None, None, cin, q), lambda i, j: (i, j, 0, 0)),
            pl.BlockSpec((dg, cin), lambda i, j: (0, 0)),
            pl.BlockSpec((dg, 1), lambda i, j: (0, 0)),
            pl.BlockSpec((dg, cin), lambda i, j: (0, 0)),
            pl.BlockSpec((dg, 1), lambda i, j: (0, 0)),
        ],
        out_specs=pl.BlockSpec((None, _TB, _V, _V), lambda i, j: (i, j, 0, 0)),
        compiler_params=pltpu.CompilerParams(
            dimension_semantics=("parallel", "parallel")),
    )(xq, wg1, bg1.reshape(dg, 1), wg2, bg2.reshape(dg, 1))


def _mix_kernel(x_ref, w_ref, c_ref, o_ref, *, relu):
    acc = jnp.dot(w_ref[...], x_ref[...], preferred_element_type=jnp.float32,
                  precision=_PREC)
    acc = acc + c_ref[...]
    if relu:
        acc = jnp.maximum(acc, 0.0)
    o_ref[...] = acc.astype(o_ref.dtype)


def _mix_res_kernel(x_ref, w_ref, c_ref, r_ref, o_ref, *, relu):
    acc = jnp.dot(w_ref[...], x_ref[...], preferred_element_type=jnp.float32,
                  precision=_PREC)
    acc = acc + c_ref[...] + r_ref[...].astype(jnp.float32)
    if relu:
        acc = jnp.maximum(acc, 0.0)
    o_ref[...] = acc.astype(o_ref.dtype)


def _mix(xq, w, c, res=None, relu=False):
    n, tc, cin, q = xq.shape
    cout = w.shape[0]
    in_specs = [
        pl.BlockSpec((None, None, cin, q), lambda i, j: (i, j, 0, 0)),
        pl.BlockSpec((cout, cin), lambda i, j: (0, 0)),
        pl.BlockSpec((cout, 1), lambda i, j: (0, 0)),
    ]
    args = [xq, w.astype(jnp.float32), c.reshape(cout, 1).astype(jnp.float32)]
    if res is None:
        kern = functools.partial(_mix_kernel, relu=relu)
    else:
        kern = functools.partial(_mix_res_kernel, relu=relu)
        in_specs.append(pl.BlockSpec((None, None, cout, q),
                                     lambda i, j: (i, j, 0, 0)))
        args.append(res)
    return pl.pallas_call(
        kern,
        out_shape=jax.ShapeDtypeStruct((n, tc, cout, q), xq.dtype),
        grid=(n, tc),
        in_specs=in_specs,
        out_specs=pl.BlockSpec((None, None, cout, q), lambda i, j: (i, j, 0, 0)),
        compiler_params=pltpu.CompilerParams(
            dimension_semantics=("parallel", "parallel")),
    )(*args)


def _attn_body(h, g_ref, ww_ref, ww1_ref, c_ref, o_ref):
    rows = []
    for tt in range(_TB):
        blk = g_ref[tt].astype(jnp.float32)
        left = tt * _V
        right = (_TB - 1 - tt) * _V
        if left:
            blk = jnp.concatenate(
                [jnp.zeros((_V, left), jnp.float32), blk], axis=1)
        if right:
            blk = jnp.concatenate(
                [blk, jnp.zeros((_V, right), jnp.float32)], axis=1)
        rows.append(blk)
    gbig = jnp.concatenate(rows, axis=0)
    a = lax.dot_general(h, gbig, (((1,), (1,)), ((), ())),
                        preferred_element_type=jnp.float32, precision=_PREC)
    out = (jnp.dot(ww_ref[...], a, preferred_element_type=jnp.float32,
                   precision=_PREC)
           + jnp.dot(ww1_ref[...], h, preferred_element_type=jnp.float32,
                     precision=_PREC)
           + c_ref[...])
    o_ref[...] = jnp.maximum(out, 0.0).astype(o_ref.dtype)


def _attn_kernel(h_ref, g_ref, ww_ref, ww1_ref, c_ref, o_ref):
    _attn_body(h_ref[...].astype(jnp.float32),
               g_ref, ww_ref, ww1_ref, c_ref, o_ref)


def _attn_res_kernel(y_ref, r_ref, g_ref, ww_ref, ww1_ref, c_ref, o_ref):
    h = jnp.maximum(y_ref[...].astype(jnp.float32)
                    + r_ref[...].astype(jnp.float32), 0.0)
    _attn_body(h, g_ref, ww_ref, ww1_ref, c_ref, o_ref)


def _attn(ysq, resq, g, ww, ww1, cc):
    n, tc, d, q = ysq.shape
    in_specs = [pl.BlockSpec((None, None, d, q), lambda i, j: (i, j, 0, 0))]
    args = [ysq]
    if resq is not None:
        in_specs.append(pl.BlockSpec((None, None, d, q),
                                     lambda i, j: (i, j, 0, 0)))
        args.append(resq)
        kern = _attn_res_kernel
    else:
        kern = _attn_kernel
    in_specs += [
        pl.BlockSpec((None, _TB, _V, _V), lambda i, j: (i, j, 0, 0)),
        pl.BlockSpec((d, d), lambda i, j: (0, 0)),
        pl.BlockSpec((d, d), lambda i, j: (0, 0)),
        pl.BlockSpec((d, 1), lambda i, j: (0, 0)),
    ]
    args += [g, ww.astype(jnp.float32), ww1.astype(jnp.float32),
             cc.reshape(d, 1).astype(jnp.float32)]
    return pl.pallas_call(
        kern,
        out_shape=jax.ShapeDtypeStruct((n, tc, d, q), ysq.dtype),
        grid=(n, tc),
        in_specs=in_specs,
        out_specs=pl.BlockSpec((None, None, d, q), lambda i, j: (i, j, 0, 0)),
        compiler_params=pltpu.CompilerParams(
            dimension_semantics=("parallel", "parallel")),
    )(*args)


def _tcn_kernel(h_ref, x_ref, wt_ref, wr_ref, c_ref, o_ref, *, cout, taps):
    hf = h_ref[...]
    acc = jnp.dot(wr_ref[...], x_ref[...],
                  preferred_element_type=jnp.float32, precision=_PREC)
    for k in range(taps):
        s = (k - (taps - 1) // 2) * _V
        if s > 0:
            xk = jnp.concatenate(
                [hf[:, s:], jnp.zeros((cout, s), hf.dtype)], axis=1)
        elif s < 0:
            xk = jnp.concatenate(
                [jnp.zeros((cout, -s), hf.dtype), hf[:, :s]], axis=1)
        else:
            xk = hf
        acc = acc + jnp.dot(wt_ref[k], xk,
                            preferred_element_type=jnp.float32,
                            precision=_PREC)
    acc = acc + c_ref[...]
    o_ref[...] = jnp.maximum(acc, 0.0).astype(o_ref.dtype)


def _tcn(hf, xf, wt, wr, ctot):
    n, cout, m = hf.shape
    cin = xf.shape[1]
    taps = wt.shape[0]
    kern = functools.partial(_tcn_kernel, cout=cout, taps=taps)
    return pl.pallas_call(
        kern,
        out_shape=jax.ShapeDtypeStruct((n, cout, m), hf.dtype),
        grid=(n,),
        in_specs=[
            pl.BlockSpec((None, cout, m), lambda i: (i, 0, 0)),
            pl.BlockSpec((None, cin, m), lambda i: (i, 0, 0)),
            pl.BlockSpec((taps, cout, cout), lambda i: (0, 0, 0)),
            pl.BlockSpec((cout, cin), lambda i: (0, 0)),
            pl.BlockSpec((cout, 1), lambda i: (0, 0)),
        ],
        out_specs=pl.BlockSpec((None, cout, m), lambda i: (i, 0, 0)),
        compiler_params=pltpu.CompilerParams(
            dimension_semantics=("parallel",)),
    )(hf, xf, wt.astype(jnp.float32), wr.astype(jnp.float32),
      ctot.reshape(cout, 1).astype(jnp.float32))


def _gcn_layer(x0q, g, Lw, Lb, FM, bn1, Ww, Ww1, bw1, bns, down):
    n, tc, c, q = x0q.shape
    d = Lw.shape[1]
    x5 = x0q.reshape(n, tc, c, _TB, _V)
    mask_cv = jnp.tanh(FM[0]).T + 1.0
    xs = jnp.concatenate(
        [jnp.roll(x5[:, :, cc:cc + 1], -cc, axis=-1) for cc in range(c)],
        axis=2)
    xs = xs * mask_cv[None, None, :, None, :]
    y = _mix(xs.reshape(n, tc, c, q), jnp.transpose(Lw), Lb)
    y5 = y.reshape(n, tc, d, _TB, _V)
    s1, b1 = _bnfold(*bn1)
    ys = jnp.concatenate(
        [jnp.roll(y5[:, :, dd:dd + 1], dd, axis=-1) for dd in range(d)],
        axis=2)
    ys = (ys * s1.reshape(_V, d).T[None, None, :, None, :]
          + b1.reshape(_V, d).T[None, None, :, None, :])
    ysq = ys.reshape(n, tc, d, q)
    ss, bs = _bnfold(*bns)
    ww = Ww * ss[:, None]
    ww1 = Ww1 * ss[:, None]
    cc = ss * bw1 + bs
    if down is None:
        return _attn(ysq, x0q, g, ww, ww1, cc)
    dw, db, dbn = down
    sd, bd = _bnfold(*dbn)
    h = _mix(x0q, dw * sd[:, None], sd * db + bd, res=ysq, relu=True)
    return _attn(h, None, g, ww, ww1, cc)


def kernel(x, g1_w, g1_b, g2_w, g2_b,
           l1_Lw, l1_Lb, l1_FM, l1_bn1_g, l1_bn1_b, l1_bn1_m, l1_bn1_v,
           l1_Ww, l1_Ww1, l1_bw1, l1_bns_g, l1_bns_b, l1_bns_m, l1_bns_v,
           l2_Lw, l2_Lb, l2_FM, l2_bn1_g, l2_bn1_b, l2_bn1_m, l2_bn1_v,
           l2_Ww, l2_Ww1, l2_bw1, l2_bns_g, l2_bns_b, l2_bns_m, l2_bns_v,
           l2_dw, l2_db, l2_dbn_g, l2_dbn_b, l2_dbn_m, l2_dbn_v,
           l3_Lw, l3_Lb, l3_FM, l3_bn1_g, l3_bn1_b, l3_bn1_m, l3_bn1_v,
           l3_Ww, l3_Ww1, l3_bw1, l3_bns_g, l3_bns_b, l3_bns_m, l3_bns_v,
           t_w, t_b, t_bn_g, t_bn_b, t_bn_m, t_bn_v,
           r_w, r_b, r_bn_g, r_bn_b, r_bn_m, r_bn_v):
    n, c, t, v = x.shape
    m = t * v
    tc = t // _TB
    xq = jnp.swapaxes(x.reshape(n, c, tc, _Q), 1, 2)

    g = _compute_g(xq, g1_w, g1_b, g2_w, g2_b, t)

    h = _gcn_layer(xq, g, l1_Lw, l1_Lb, l1_FM,
                   (l1_bn1_g, l1_bn1_b, l1_bn1_m, l1_bn1_v),
                   l1_Ww, l1_Ww1, l1_bw1,
                   (l1_bns_g, l1_bns_b, l1_bns_m, l1_bns_v), None)
    h = _gcn_layer(h, g, l2_Lw, l2_Lb, l2_FM,
                   (l2_bn1_g, l2_bn1_b, l2_bn1_m, l2_bn1_v),
                   l2_Ww, l2_Ww1, l2_bw1,
                   (l2_bns_g, l2_bns_b, l2_bns_m, l2_bns_v),
                   (l2_dw, l2_db, (l2_dbn_g, l2_dbn_b, l2_dbn_m, l2_dbn_v)))
    h = _gcn_layer(h, g, l3_Lw, l3_Lb, l3_FM,
                   (l3_bn1_g, l3_bn1_b, l3_bn1_m, l3_bn1_v),
                   l3_Ww, l3_Ww1, l3_bw1,
                   (l3_bns_g, l3_bns_b, l3_bns_m, l3_bns_v), None)

    cout = h.shape[2]
    hf = jnp.swapaxes(h, 1, 2).reshape(n, cout, m)
    sr, br = _bnfold(r_bn_g, r_bn_b, r_bn_m, r_bn_v)
    wr = r_w[:, :, 0] * sr[:, None]
    cr = sr * r_b + br
    st, bt = _bnfold(t_bn_g, t_bn_b, t_bn_m, t_bn_v)
    wt = jnp.transpose(t_w, (2, 0, 1)) * st[None, :, None]
    ctot = st * t_b + bt + cr
    out = _tcn(hf, x.reshape(n, c, m), wt, wr, ctot)
    return out.reshape(n, cout, t, v)


# speedup vs baseline: 14.2977x; 14.2977x over previous
import functools

import jax
import jax.numpy as jnp
from jax import lax
from jax.experimental import pallas as pl
from jax.experimental.pallas import tpu as pltpu

_EPS = 1e-5
_V = 25
_TB = 8
_Q = _TB * _V
_PREC = lax.Precision.DEFAULT
_PREC_G = lax.Precision.HIGHEST


def _bnfold(g, b, m, v):
    s = g / jnp.sqrt(v + _EPS)
    return s, b - s * m


def _g_kernel(x_ref, w1_ref, b1_ref, w2_ref, b2_ref, g_ref, *, tc):
    w1 = w1_ref[...]
    b1 = b1_ref[...]
    w2 = w2_ref[...]
    b2 = b2_ref[...]
    for k in range(tc):
        x = x_ref[k]
        p = jnp.dot(w1, x, preferred_element_type=jnp.float32,
                    precision=_PREC_G) + b1
        q = jnp.dot(w2, x, preferred_element_type=jnp.float32,
                    precision=_PREC_G) + b2
        s = lax.dot_general(p, q, (((0,), (0,)), ((), ())),
                            preferred_element_type=jnp.float32,
                            precision=_PREC_G)
        for tt in range(_TB):
            sb = s[tt * _V:(tt + 1) * _V, tt * _V:(tt + 1) * _V]
            sb = sb - jnp.max(sb, axis=-1, keepdims=True)
            e = jnp.exp(sb)
            g_ref[k * _TB + tt] = (
                e / jnp.sum(e, axis=-1, keepdims=True)).astype(g_ref.dtype)


def _compute_g(xq, wg1, bg1, wg2, bg2, t):
    n, tc, cin, q = xq.shape
    dg = wg1.shape[0]
    return pl.pallas_call(
        functools.partial(_g_kernel, tc=tc),
        out_shape=jax.ShapeDtypeStruct((n, t, _V, _V), xq.dtype),
        grid=(n,),
        in_specs=[
            pl.BlockSpec((None, tc, cin, q), lambda i: (i, 0, 0, 0)),
            pl.BlockSpec((dg, cin), lambda i: (0, 0)),
            pl.BlockSpec((dg, 1), lambda i: (0, 0)),
            pl.BlockSpec((dg, cin), lambda i: (0, 0)),
            pl.BlockSpec((dg, 1), lambda i: (0, 0)),
        ],
        out_specs=pl.BlockSpec((None, t, _V, _V), lambda i: (i, 0, 0, 0)),
        compiler_params=pltpu.CompilerParams(
            dimension_semantics=("parallel",)),
    )(xq, wg1, bg1.reshape(dg, 1), wg2, bg2.reshape(dg, 1))


def _mix_kernel(x_ref, w_ref, c_ref, o_ref, *, relu, tc):
    w = w_ref[...]
    cc = c_ref[...]
    for k in range(tc):
        acc = jnp.dot(w, x_ref[k], preferred_element_type=jnp.float32,
                      precision=_PREC)
        acc = acc + cc
        if relu:
            acc = jnp.maximum(acc, 0.0)
        o_ref[k] = acc.astype(o_ref.dtype)


def _mix_res_kernel(x_ref, w_ref, c_ref, r_ref, o_ref, *, relu, tc):
    w = w_ref[...]
    cc = c_ref[...]
    for k in range(tc):
        acc = jnp.dot(w, x_ref[k], preferred_element_type=jnp.float32,
                      precision=_PREC)
        acc = acc + cc + r_ref[k].astype(jnp.float32)
        if relu:
            acc = jnp.maximum(acc, 0.0)
        o_ref[k] = acc.astype(o_ref.dtype)


def _mix(xq, w, c, res=None, relu=False):
    n, tc, cin, q = xq.shape
    cout = w.shape[0]
    in_specs = [
        pl.BlockSpec((None, tc, cin, q), lambda i: (i, 0, 0, 0)),
        pl.BlockSpec((cout, cin), lambda i: (0, 0)),
        pl.BlockSpec((cout, 1), lambda i: (0, 0)),
    ]
    args = [xq, w.astype(jnp.float32), c.reshape(cout, 1).astype(jnp.float32)]
    if res is None:
        kern = functools.partial(_mix_kernel, relu=relu, tc=tc)
    else:
        kern = functools.partial(_mix_res_kernel, relu=relu, tc=tc)
        in_specs.append(pl.BlockSpec((None, tc, cout, q),
                                     lambda i: (i, 0, 0, 0)))
        args.append(res)
    return pl.pallas_call(
        kern,
        out_shape=jax.ShapeDtypeStruct((n, tc, cout, q), xq.dtype),
        grid=(n,),
        in_specs=in_specs,
        out_specs=pl.BlockSpec((None, tc, cout, q), lambda i: (i, 0, 0, 0)),
        compiler_params=pltpu.CompilerParams(
            dimension_semantics=("parallel",)),
    )(*args)


def _attn_chunk(h, g_ref, k, ww, ww1, cc, o_ref):
    rows = []
    for tt in range(_TB):
        blk = g_ref[k * _TB + tt].astype(jnp.float32)
        left = tt * _V
        right = (_TB - 1 - tt) * _V
        if left:
            blk = jnp.concatenate(
                [jnp.zeros((_V, left), jnp.float32), blk], axis=1)
        if right:
            blk = jnp.concatenate(
                [blk, jnp.zeros((_V, right), jnp.float32)], axis=1)
        rows.append(blk)
    gbig = jnp.concatenate(rows, axis=0)
    a = lax.dot_general(h, gbig, (((1,), (1,)), ((), ())),
                        preferred_element_type=jnp.float32, precision=_PREC)
    out = (jnp.dot(ww, a, preferred_element_type=jnp.float32,
                   precision=_PREC)
           + jnp.dot(ww1, h, preferred_element_type=jnp.float32,
                     precision=_PREC)
           + cc)
    o_ref[k] = jnp.maximum(out, 0.0).astype(o_ref.dtype)


def _attn_kernel(h_ref, g_ref, ww_ref, ww1_ref, c_ref, o_ref, *, tc):
    ww = ww_ref[...]
    ww1 = ww1_ref[...]
    cc = c_ref[...]
    for k in range(tc):
        _attn_chunk(h_ref[k].astype(jnp.float32), g_ref, k, ww, ww1, cc, o_ref)


def _attn_res_kernel(y_ref, r_ref, g_ref, ww_ref, ww1_ref, c_ref, o_ref, *, tc):
    ww = ww_ref[...]
    ww1 = ww1_ref[...]
    cc = c_ref[...]
    for k in range(tc):
        h = jnp.maximum(y_ref[k].astype(jnp.float32)
                        + r_ref[k].astype(jnp.float32), 0.0)
        _attn_chunk(h, g_ref, k, ww, ww1, cc, o_ref)


def _attn(ysq, resq, g, ww, ww1, cc):
    n, tc, d, q = ysq.shape
    t = g.shape[1]
    in_specs = [pl.BlockSpec((None, tc, d, q), lambda i: (i, 0, 0, 0))]
    args = [ysq]
    if resq is not None:
        in_specs.append(pl.BlockSpec((None, tc, d, q),
                                     lambda i: (i, 0, 0, 0)))
        args.append(resq)
        kern = functools.partial(_attn_res_kernel, tc=tc)
    else:
        kern = functools.partial(_attn_kernel, tc=tc)
    in_specs += [
        pl.BlockSpec((None, t, _V, _V), lambda i: (i, 0, 0, 0)),
        pl.BlockSpec((d, d), lambda i: (0, 0)),
        pl.BlockSpec((d, d), lambda i: (0, 0)),
        pl.BlockSpec((d, 1), lambda i: (0, 0)),
    ]
    args += [g, ww.astype(jnp.float32), ww1.astype(jnp.float32),
             cc.reshape(d, 1).astype(jnp.float32)]
    return pl.pallas_call(
        kern,
        out_shape=jax.ShapeDtypeStruct((n, tc, d, q), ysq.dtype),
        grid=(n,),
        in_specs=in_specs,
        out_specs=pl.BlockSpec((None, tc, d, q), lambda i: (i, 0, 0, 0)),
        compiler_params=pltpu.CompilerParams(
            dimension_semantics=("parallel",)),
    )(*args)


def _tcn_kernel(h_ref, x_ref, wt_ref, wr_ref, c_ref, o_ref, *, cout, taps):
    hf = h_ref[...]
    acc = jnp.dot(wr_ref[...], x_ref[...],
                  preferred_element_type=jnp.float32, precision=_PREC)
    for k in range(taps):
        s = (k - (taps - 1) // 2) * _V
        if s > 0:
            xk = jnp.concatenate(
                [hf[:, s:], jnp.zeros((cout, s), hf.dtype)], axis=1)
        elif s < 0:
            xk = jnp.concatenate(
                [jnp.zeros((cout, -s), hf.dtype), hf[:, :s]], axis=1)
        else:
            xk = hf
        acc = acc + jnp.dot(wt_ref[k], xk,
                            preferred_element_type=jnp.float32,
                            precision=_PREC)
    acc = acc + c_ref[...]
    o_ref[...] = jnp.maximum(acc, 0.0).astype(o_ref.dtype)


def _tcn(hf, xf, wt, wr, ctot):
    n, cout, m = hf.shape
    cin = xf.shape[1]
    taps = wt.shape[0]
    kern = functools.partial(_tcn_kernel, cout=cout, taps=taps)
    return pl.pallas_call(
        kern,
        out_shape=jax.ShapeDtypeStruct((n, cout, m), hf.dtype),
        grid=(n,),
        in_specs=[
            pl.BlockSpec((None, cout, m), lambda i: (i, 0, 0)),
            pl.BlockSpec((None, cin, m), lambda i: (i, 0, 0)),
            pl.BlockSpec((taps, cout, cout), lambda i: (0, 0, 0)),
            pl.BlockSpec((cout, cin), lambda i: (0, 0)),
            pl.BlockSpec((cout, 1), lambda i: (0, 0)),
        ],
        out_specs=pl.BlockSpec((None, cout, m), lambda i: (i, 0, 0)),
        compiler_params=pltpu.CompilerParams(
            dimension_semantics=("parallel",)),
    )(hf, xf, wt.astype(jnp.float32), wr.astype(jnp.float32),
      ctot.reshape(cout, 1).astype(jnp.float32))


def _gcn_layer(x0q, g, Lw, Lb, FM, bn1, Ww, Ww1, bw1, bns, down):
    n, tc, c, q = x0q.shape
    d = Lw.shape[1]
    x5 = x0q.reshape(n, tc, c, _TB, _V)
    av = jnp.arange(_V)
    mask_cv = jnp.tanh(FM[0]).T + 1.0
    pin = (av[None, :, None]
           == (av[None, None, :] + jnp.arange(c)[:, None, None]) % _V)
    pin = pin.astype(jnp.float32) * mask_cv[:, None, :]
    xs = jnp.einsum('nkctu,cuv->nkctv', x5, pin,
                    precision=lax.Precision.HIGHEST)
    y = _mix(xs.reshape(n, tc, c, q), jnp.transpose(Lw), Lb)
    y5 = y.reshape(n, tc, d, _TB, _V)
    s1, b1 = _bnfold(*bn1)
    s1_dv = s1.reshape(_V, d).T
    b1_dv = b1.reshape(_V, d).T
    pout = (av[None, :, None]
            == (av[None, None, :] - jnp.arange(d)[:, None, None]) % _V)
    pout = pout.astype(jnp.float32) * s1_dv[:, None, :]
    ys = (jnp.einsum('nkdtu,duv->nkdtv', y5, pout,
                     precision=lax.Precision.HIGHEST)
          + b1_dv[None, None, :, None, :])
    ysq = ys.reshape(n, tc, d, q)
    ss, bs = _bnfold(*bns)
    ww = Ww * ss[:, None]
    ww1 = Ww1 * ss[:, None]
    cc = ss * bw1 + bs
    if down is None:
        return _attn(ysq, x0q, g, ww, ww1, cc)
    dw, db, dbn = down
    sd, bd = _bnfold(*dbn)
    h = _mix(x0q, dw * sd[:, None], sd * db + bd, res=ysq, relu=True)
    return _attn(h, None, g, ww, ww1, cc)


def kernel(x, g1_w, g1_b, g2_w, g2_b,
           l1_Lw, l1_Lb, l1_FM, l1_bn1_g, l1_bn1_b, l1_bn1_m, l1_bn1_v,
           l1_Ww, l1_Ww1, l1_bw1, l1_bns_g, l1_bns_b, l1_bns_m, l1_bns_v,
           l2_Lw, l2_Lb, l2_FM, l2_bn1_g, l2_bn1_b, l2_bn1_m, l2_bn1_v,
           l2_Ww, l2_Ww1, l2_bw1, l2_bns_g, l2_bns_b, l2_bns_m, l2_bns_v,
           l2_dw, l2_db, l2_dbn_g, l2_dbn_b, l2_dbn_m, l2_dbn_v,
           l3_Lw, l3_Lb, l3_FM, l3_bn1_g, l3_bn1_b, l3_bn1_m, l3_bn1_v,
           l3_Ww, l3_Ww1, l3_bw1, l3_bns_g, l3_bns_b, l3_bns_m, l3_bns_v,
           t_w, t_b, t_bn_g, t_bn_b, t_bn_m, t_bn_v,
           r_w, r_b, r_bn_g, r_bn_b, r_bn_m, r_bn_v):
    n, c, t, v = x.shape
    m = t * v
    tc = t // _TB
    xq = jnp.swapaxes(x.reshape(n, c, tc, _Q), 1, 2)

    g = _compute_g(xq, g1_w, g1_b, g2_w, g2_b, t)

    h = _gcn_layer(xq, g, l1_Lw, l1_Lb, l1_FM,
                   (l1_bn1_g, l1_bn1_b, l1_bn1_m, l1_bn1_v),
                   l1_Ww, l1_Ww1, l1_bw1,
                   (l1_bns_g, l1_bns_b, l1_bns_m, l1_bns_v), None)
    h = _gcn_layer(h, g, l2_Lw, l2_Lb, l2_FM,
                   (l2_bn1_g, l2_bn1_b, l2_bn1_m, l2_bn1_v),
                   l2_Ww, l2_Ww1, l2_bw1,
                   (l2_bns_g, l2_bns_b, l2_bns_m, l2_bns_v),
                   (l2_dw, l2_db, (l2_dbn_g, l2_dbn_b, l2_dbn_m, l2_dbn_v)))
    h = _gcn_layer(h, g, l3_Lw, l3_Lb, l3_FM,
                   (l3_bn1_g, l3_bn1_b, l3_bn1_m, l3_bn1_v),
                   l3_Ww, l3_Ww1, l3_bw1,
                   (l3_bns_g, l3_bns_b, l3_bns_m, l3_bns_v), None)

    cout = h.shape[2]
    hf = jnp.swapaxes(h, 1, 2).reshape(n, cout, m)
    sr, br = _bnfold(r_bn_g, r_bn_b, r_bn_m, r_bn_v)
    wr = r_w[:, :, 0] * sr[:, None]
    cr = sr * r_b + br
    st, bt = _bnfold(t_bn_g, t_bn_b, t_bn_m, t_bn_v)
    wt = jnp.transpose(t_w, (2, 0, 1)) * st[None, :, None]
    ctot = st * t_b + bt + cr
    out = _tcn(hf, x.reshape(n, c, m), wt, wr, ctot)
    return out.reshape(n, cout, t, v)


# speedup vs baseline: 17.9611x; 1.2562x over previous
import functools

import jax
import jax.numpy as jnp
from jax import lax
from jax.experimental import pallas as pl
from jax.experimental.pallas import tpu as pltpu

_EPS = 1e-5
_V = 25
_TB = 8
_Q = _TB * _V
_PREC = lax.Precision.DEFAULT
_PREC_G = lax.Precision.HIGHEST


def _bnfold(g, b, m, v):
    s = g / jnp.sqrt(v + _EPS)
    return s, b - s * m


def _g_kernel(x_ref, w1_ref, b1_ref, w2_ref, b2_ref, m_ref, g_ref, *, tc):
    w1 = w1_ref[...]
    b1 = b1_ref[...]
    w2 = w2_ref[...]
    b2 = b2_ref[...]
    mask = m_ref[...]
    for k in range(tc):
        x = x_ref[k]
        p = jnp.dot(w1, x, preferred_element_type=jnp.float32,
                    precision=_PREC_G) + b1
        q = jnp.dot(w2, x, preferred_element_type=jnp.float32,
                    precision=_PREC_G) + b2
        s = lax.dot_general(p, q, (((0,), (0,)), ((), ())),
                            preferred_element_type=jnp.float32,
                            precision=_PREC_G) + mask
        s = s - jnp.max(s, axis=-1, keepdims=True)
        e = jnp.exp(s)
        g_ref[k] = (e / jnp.sum(e, axis=-1, keepdims=True)).astype(g_ref.dtype)


def _compute_g(xq, wg1, bg1, wg2, bg2):
    n, tc, cin, q = xq.shape
    dg = wg1.shape[0]
    aq = jnp.arange(q) // _V
    mask = jnp.where(aq[:, None] == aq[None, :], 0.0, -1e30).astype(jnp.float32)
    return pl.pallas_call(
        functools.partial(_g_kernel, tc=tc),
        out_shape=jax.ShapeDtypeStruct((n, tc, q, q), xq.dtype),
        grid=(n,),
        in_specs=[
            pl.BlockSpec((None, tc, cin, q), lambda i: (i, 0, 0, 0)),
            pl.BlockSpec((dg, cin), lambda i: (0, 0)),
            pl.BlockSpec((dg, 1), lambda i: (0, 0)),
            pl.BlockSpec((dg, cin), lambda i: (0, 0)),
            pl.BlockSpec((dg, 1), lambda i: (0, 0)),
            pl.BlockSpec((q, q), lambda i: (0, 0)),
        ],
        out_specs=pl.BlockSpec((None, tc, q, q), lambda i: (i, 0, 0, 0)),
        compiler_params=pltpu.CompilerParams(
            dimension_semantics=("parallel",)),
    )(xq, wg1, bg1.reshape(dg, 1), wg2, bg2.reshape(dg, 1), mask)


def _mix_kernel(x_ref, w_ref, c_ref, o_ref, *, relu, tc):
    w = w_ref[...]
    cc = c_ref[...]
    for k in range(tc):
        acc = jnp.dot(w, x_ref[k], preferred_element_type=jnp.float32,
                      precision=_PREC)
        acc = acc + cc
        if relu:
            acc = jnp.maximum(acc, 0.0)
        o_ref[k] = acc.astype(o_ref.dtype)


def _mix_res_kernel(x_ref, w_ref, c_ref, r_ref, o_ref, *, relu, tc):
    w = w_ref[...]
    cc = c_ref[...]
    for k in range(tc):
        acc = jnp.dot(w, x_ref[k], preferred_element_type=jnp.float32,
                      precision=_PREC)
        acc = acc + cc + r_ref[k].astype(jnp.float32)
        if relu:
            acc = jnp.maximum(acc, 0.0)
        o_ref[k] = acc.astype(o_ref.dtype)


def _mix(xq, w, c, res=None, relu=False):
    n, tc, cin, q = xq.shape
    cout = w.shape[0]
    in_specs = [
        pl.BlockSpec((None, tc, cin, q), lambda i: (i, 0, 0, 0)),
        pl.BlockSpec((cout, cin), lambda i: (0, 0)),
        pl.BlockSpec((cout, 1), lambda i: (0, 0)),
    ]
    args = [xq, w.astype(jnp.float32), c.reshape(cout, 1).astype(jnp.float32)]
    if res is None:
        kern = functools.partial(_mix_kernel, relu=relu, tc=tc)
    else:
        kern = functools.partial(_mix_res_kernel, relu=relu, tc=tc)
        in_specs.append(pl.BlockSpec((None, tc, cout, q),
                                     lambda i: (i, 0, 0, 0)))
        args.append(res)
    return pl.pallas_call(
        kern,
        out_shape=jax.ShapeDtypeStruct((n, tc, cout, q), xq.dtype),
        grid=(n,),
        in_specs=in_specs,
        out_specs=pl.BlockSpec((None, tc, cout, q), lambda i: (i, 0, 0, 0)),
        compiler_params=pltpu.CompilerParams(
            dimension_semantics=("parallel",)),
    )(*args)


def _attn_chunk(h, g_ref, k, ww, ww1, cc, o_ref):
    gbig = g_ref[k]
    a = lax.dot_general(h, gbig, (((1,), (1,)), ((), ())),
                        preferred_element_type=jnp.float32, precision=_PREC)
    out = (jnp.dot(ww, a, preferred_element_type=jnp.float32,
                   precision=_PREC)
           + jnp.dot(ww1, h, preferred_element_type=jnp.float32,
                     precision=_PREC)
           + cc)
    o_ref[k] = jnp.maximum(out, 0.0).astype(o_ref.dtype)


def _attn_kernel(h_ref, g_ref, ww_ref, ww1_ref, c_ref, o_ref, *, tc):
    ww = ww_ref[...]
    ww1 = ww1_ref[...]
    cc = c_ref[...]
    for k in range(tc):
        _attn_chunk(h_ref[k].astype(jnp.float32), g_ref, k, ww, ww1, cc, o_ref)


def _attn_res_kernel(y_ref, r_ref, g_ref, ww_ref, ww1_ref, c_ref, o_ref, *, tc):
    ww = ww_ref[...]
    ww1 = ww1_ref[...]
    cc = c_ref[...]
    for k in range(tc):
        h = jnp.maximum(y_ref[k].astype(jnp.float32)
                        + r_ref[k].astype(jnp.float32), 0.0)
        _attn_chunk(h, g_ref, k, ww, ww1, cc, o_ref)


def _attn(ysq, resq, g, ww, ww1, cc):
    n, tc, d, q = ysq.shape
    in_specs = [pl.BlockSpec((None, tc, d, q), lambda i: (i, 0, 0, 0))]
    args = [ysq]
    if resq is not None:
        in_specs.append(pl.BlockSpec((None, tc, d, q),
                                     lambda i: (i, 0, 0, 0)))
        args.append(resq)
        kern = functools.partial(_attn_res_kernel, tc=tc)
    else:
        kern = functools.partial(_attn_kernel, tc=tc)
    in_specs += [
        pl.BlockSpec((None, tc, q, q), lambda i: (i, 0, 0, 0)),
        pl.BlockSpec((d, d), lambda i: (0, 0)),
        pl.BlockSpec((d, d), lambda i: (0, 0)),
        pl.BlockSpec((d, 1), lambda i: (0, 0)),
    ]
    args += [g, ww.astype(jnp.float32), ww1.astype(jnp.float32),
             cc.reshape(d, 1).astype(jnp.float32)]
    return pl.pallas_call(
        kern,
        out_shape=jax.ShapeDtypeStruct((n, tc, d, q), ysq.dtype),
        grid=(n,),
        in_specs=in_specs,
        out_specs=pl.BlockSpec((None, tc, d, q), lambda i: (i, 0, 0, 0)),
        compiler_params=pltpu.CompilerParams(
            dimension_semantics=("parallel",)),
    )(*args)


def _tcn_kernel(h_ref, x_ref, wt_ref, wr_ref, c_ref, o_ref, *, cout, taps):
    hf = h_ref[...]
    acc = jnp.dot(wr_ref[...], x_ref[...],
                  preferred_element_type=jnp.float32, precision=_PREC)
    for k in range(taps):
        s = (k - (taps - 1) // 2) * _V
        if s > 0:
            xk = jnp.concatenate(
                [hf[:, s:], jnp.zeros((cout, s), hf.dtype)], axis=1)
        elif s < 0:
            xk = jnp.concatenate(
                [jnp.zeros((cout, -s), hf.dtype), hf[:, :s]], axis=1)
        else:
            xk = hf
        acc = acc + jnp.dot(wt_ref[k], xk,
                            preferred_element_type=jnp.float32,
                            precision=_PREC)
    acc = acc + c_ref[...]
    o_ref[...] = jnp.maximum(acc, 0.0).astype(o_ref.dtype)


def _tcn(hf, xf, wt, wr, ctot):
    n, cout, m = hf.shape
    cin = xf.shape[1]
    taps = wt.shape[0]
    kern = functools.partial(_tcn_kernel, cout=cout, taps=taps)
    return pl.pallas_call(
        kern,
        out_shape=jax.ShapeDtypeStruct((n, cout, m), hf.dtype),
        grid=(n,),
        in_specs=[
            pl.BlockSpec((None, cout, m), lambda i: (i, 0, 0)),
            pl.BlockSpec((None, cin, m), lambda i: (i, 0, 0)),
            pl.BlockSpec((taps, cout, cout), lambda i: (0, 0, 0)),
            pl.BlockSpec((cout, cin), lambda i: (0, 0)),
            pl.BlockSpec((cout, 1), lambda i: (0, 0)),
        ],
        out_specs=pl.BlockSpec((None, cout, m), lambda i: (i, 0, 0)),
        compiler_params=pltpu.CompilerParams(
            dimension_semantics=("parallel",)),
    )(hf, xf, wt.astype(jnp.float32), wr.astype(jnp.float32),
      ctot.reshape(cout, 1).astype(jnp.float32))


def _gcn_layer(x0q, g, Lw, Lb, FM, bn1, Ww, Ww1, bw1, bns, down):
    n, tc, c, q = x0q.shape
    d = Lw.shape[1]
    x5 = x0q.reshape(n, tc, c, _TB, _V)
    av = jnp.arange(_V)
    mask_cv = jnp.tanh(FM[0]).T + 1.0
    pin = (av[None, :, None]
           == (av[None, None, :] + jnp.arange(c)[:, None, None]) % _V)
    pin = pin.astype(jnp.float32) * mask_cv[:, None, :]
    xs = jnp.einsum('nkctu,cuv->nkctv', x5, pin,
                    precision=lax.Precision.HIGHEST)
    y = _mix(xs.reshape(n, tc, c, q), jnp.transpose(Lw), Lb)
    y5 = y.reshape(n, tc, d, _TB, _V)
    s1, b1 = _bnfold(*bn1)
    s1_dv = s1.reshape(_V, d).T
    b1_dv = b1.reshape(_V, d).T
    pout = (av[None, :, None]
            == (av[None, None, :] - jnp.arange(d)[:, None, None]) % _V)
    pout = pout.astype(jnp.float32) * s1_dv[:, None, :]
    ys = (jnp.einsum('nkdtu,duv->nkdtv', y5, pout,
                     precision=lax.Precision.HIGHEST)
          + b1_dv[None, None, :, None, :])
    ysq = ys.reshape(n, tc, d, q)
    ss, bs = _bnfold(*bns)
    ww = Ww * ss[:, None]
    ww1 = Ww1 * ss[:, None]
    cc = ss * bw1 + bs
    if down is None:
        return _attn(ysq, x0q, g, ww, ww1, cc)
    dw, db, dbn = down
    sd, bd = _bnfold(*dbn)
    h = _mix(x0q, dw * sd[:, None], sd * db + bd, res=ysq, relu=True)
    return _attn(h, None, g, ww, ww1, cc)


def kernel(x, g1_w, g1_b, g2_w, g2_b,
           l1_Lw, l1_Lb, l1_FM, l1_bn1_g, l1_bn1_b, l1_bn1_m, l1_bn1_v,
           l1_Ww, l1_Ww1, l1_bw1, l1_bns_g, l1_bns_b, l1_bns_m, l1_bns_v,
           l2_Lw, l2_Lb, l2_FM, l2_bn1_g, l2_bn1_b, l2_bn1_m, l2_bn1_v,
           l2_Ww, l2_Ww1, l2_bw1, l2_bns_g, l2_bns_b, l2_bns_m, l2_bns_v,
           l2_dw, l2_db, l2_dbn_g, l2_dbn_b, l2_dbn_m, l2_dbn_v,
           l3_Lw, l3_Lb, l3_FM, l3_bn1_g, l3_bn1_b, l3_bn1_m, l3_bn1_v,
           l3_Ww, l3_Ww1, l3_bw1, l3_bns_g, l3_bns_b, l3_bns_m, l3_bns_v,
           t_w, t_b, t_bn_g, t_bn_b, t_bn_m, t_bn_v,
           r_w, r_b, r_bn_g, r_bn_b, r_bn_m, r_bn_v):
    n, c, t, v = x.shape
    m = t * v
    tc = t // _TB
    xq = jnp.swapaxes(x.reshape(n, c, tc, _Q), 1, 2)

    g = _compute_g(xq, g1_w, g1_b, g2_w, g2_b)

    h = _gcn_layer(xq, g, l1_Lw, l1_Lb, l1_FM,
                   (l1_bn1_g, l1_bn1_b, l1_bn1_m, l1_bn1_v),
                   l1_Ww, l1_Ww1, l1_bw1,
                   (l1_bns_g, l1_bns_b, l1_bns_m, l1_bns_v), None)
    h = _gcn_layer(h, g, l2_Lw, l2_Lb, l2_FM,
                   (l2_bn1_g, l2_bn1_b, l2_bn1_m, l2_bn1_v),
                   l2_Ww, l2_Ww1, l2_bw1,
                   (l2_bns_g, l2_bns_b, l2_bns_m, l2_bns_v),
                   (l2_dw, l2_db, (l2_dbn_g, l2_dbn_b, l2_dbn_m, l2_dbn_v)))
    h = _gcn_layer(h, g, l3_Lw, l3_Lb, l3_FM,
                   (l3_bn1_g, l3_bn1_b, l3_bn1_m, l3_bn1_v),
                   l3_Ww, l3_Ww1, l3_bw1,
                   (l3_bns_g, l3_bns_b, l3_bns_m, l3_bns_v), None)

    cout = h.shape[2]
    hf = jnp.swapaxes(h, 1, 2).reshape(n, cout, m)
    sr, br = _bnfold(r_bn_g, r_bn_b, r_bn_m, r_bn_v)
    wr = r_w[:, :, 0] * sr[:, None]
    cr = sr * r_b + br
    st, bt = _bnfold(t_bn_g, t_bn_b, t_bn_m, t_bn_v)
    wt = jnp.transpose(t_w, (2, 0, 1)) * st[None, :, None]
    ctot = st * t_b + bt + cr
    out = _tcn(hf, x.reshape(n, c, m), wt, wr, ctot)
    return out.reshape(n, cout, t, v)
